# baseline (speedup 1.0000x reference)
import functools

import jax
import jax.numpy as jnp
from jax import lax
from jax.experimental import pallas as pl
from jax.experimental.pallas import tpu as pltpu
from jax.experimental.pallas import tpu_sc as plsc

H = 1024
I = 2816
E = 9
K = 2
T = 2048
R = T * K
TM = 512
TI = 256
NI = I // TI
NB = 16
RP = NB * TM
NC, NS = 2, 16
NW = NC * NS


def _dispatch_body(x_ref, wr_ref, pos_ref, w1_ref, w2_ref, we_ref, act_ref,
                   bmap_ref):
    xr = x_ref[...]
    wr = wr_ref[...]
    logits = lax.dot_general(xr, wr, (((1,), (1,)), ((), ())),
                             preferred_element_type=jnp.float32)
    iota_e = lax.broadcasted_iota(jnp.int32, (T, E), 1)
    m1 = jnp.max(logits, axis=1, keepdims=True)
    a1 = jnp.min(jnp.where(logits == m1, iota_e, E), axis=1, keepdims=True)
    neg = jnp.where(iota_e == a1, -jnp.inf, logits)
    m2 = jnp.max(neg, axis=1, keepdims=True)
    a2 = jnp.min(jnp.where(neg == m2, iota_e, E), axis=1, keepdims=True)
    tt = jnp.exp(m2 - m1)
    w1 = 1.0 / (1.0 + tt)
    w2 = 1.0 - w1
    ev = jnp.concatenate([a1, a2], axis=0)
    oh = (ev == lax.broadcasted_iota(jnp.int32, (R, E), 1)).astype(jnp.float32)
    C = 512
    ci = lax.broadcasted_iota(jnp.int32, (C, C), 0)
    cj = lax.broadcasted_iota(jnp.int32, (C, C), 1)
    lmat = (ci > cj).astype(jnp.float32)
    off = jnp.zeros((1, E), jnp.float32)
    ranks = []
    for c in range(R // C):
        ohc = lax.slice(oh, (c * C, 0), ((c + 1) * C, E))
        loc = lax.dot_general(lmat, ohc, (((1,), (0,)), ((), ())),
                              preferred_element_type=jnp.float32)
        ranks.append(jnp.sum(ohc * (loc + off), axis=1, keepdims=True))
        off = off + lax.slice(loc + ohc, (C - 1, 0), (C, E))
    rank = jnp.concatenate(ranks, axis=0)
    counts = off
    nb = jnp.floor((counts + (TM - 1)) / TM)
    ei = lax.broadcasted_iota(jnp.int32, (E, E), 0)
    ej = lax.broadcasted_iota(jnp.int32, (E, E), 1)
    tmat = (ei < ej).astype(jnp.float32)
    esum = lax.dot_general(nb, tmat, (((1,), (0,)), ((), ())),
                           preferred_element_type=jnp.float32)
    start = esum * TM
    posf = jnp.sum(oh * start, axis=1, keepdims=True) + rank
    pos_ref[...] = posf.astype(jnp.int32)
    w1_ref[...] = jnp.broadcast_to(w1, (T, 16))
    w2_ref[...] = jnp.broadcast_to(w2, (T, 16))
    tot = lax.slice(esum + nb, (0, E - 1), (1, E))
    bio = lax.broadcasted_iota(jnp.int32, (NB, 1), 0).astype(jnp.float32)
    act = (bio < tot).astype(jnp.int32)
    bcl = jnp.minimum(bio, tot - 1.0)
    cmp = (esum <= bcl).astype(jnp.float32)
    be = jnp.sum(cmp, axis=1, keepdims=True) - 1.0
    we_ref[...] = be.astype(jnp.int32)
    act_ref[...] = act
    bmap_ref[...] = bcl.astype(jnp.int32)


def _dispatch(x2d, wr, interpret=False):
    outs = (
        jax.ShapeDtypeStruct((R, 1), jnp.int32),
        jax.ShapeDtypeStruct((T, 16), jnp.float32),
        jax.ShapeDtypeStruct((T, 16), jnp.float32),
        jax.ShapeDtypeStruct((NB, 1), jnp.int32),
        jax.ShapeDtypeStruct((NB, 1), jnp.int32),
        jax.ShapeDtypeStruct((NB, 1), jnp.int32),
    )
    return pl.pallas_call(_dispatch_body, out_shape=outs,
                          interpret=interpret)(x2d, wr)


def _ffn_body(we_s, act_s, bmap_s, x_blk, g_blk, u_blk, d_blk, y_blk, xbf):
    b = pl.program_id(0)
    i = pl.program_id(1)

    @pl.when(act_s[b] == 1)
    def _():
        @pl.when(i == 0)
        def _():
            xbf[...] = x_blk[...].astype(jnp.bfloat16)

        x = xbf[...]
        gb = g_blk[0].astype(jnp.bfloat16)
        ub = u_blk[0].astype(jnp.bfloat16)
        db = d_blk[0].astype(jnp.bfloat16)
        g = lax.dot_general(x, gb, (((1,), (1,)), ((), ())),
                            preferred_element_type=jnp.float32)
        u = lax.dot_general(x, ub, (((1,), (1,)), ((), ())),
                            preferred_element_type=jnp.float32)
        h = (g * (1.0 / (1.0 + jnp.exp(-g))) * u).astype(jnp.bfloat16)
        contrib = lax.dot_general(h, db, (((1,), (1,)), ((), ())),
                                  preferred_element_type=jnp.float32)

        @pl.when(i == 0)
        def _():
            y_blk[...] = contrib

        @pl.when(i != 0)
        def _():
            y_blk[...] = y_blk[...] + contrib


def _ffn(we, act, bmap, xs, gate_w, up_w, down_w, interpret=False):
    def xmap(b, i, we_s, act_s, bm_s):
        return (bm_s[b], 0)

    def imap(b, i, act_s):
        return jnp.where(act_s[b] == 1, i, NI - 1)

    grid_spec = pltpu.PrefetchScalarGridSpec(
        num_scalar_prefetch=3,
        grid=(NB, NI),
        in_specs=[
            pl.BlockSpec((TM, H), xmap),
            pl.BlockSpec((1, TI, H),
                         lambda b, i, we_s, act_s, bm_s:
                         (we_s[b], imap(b, i, act_s), 0)),
            pl.BlockSpec((1, TI, H),
                         lambda b, i, we_s, act_s, bm_s:
                         (we_s[b], imap(b, i, act_s), 0)),
            pl.BlockSpec((1, H, TI),
                         lambda b, i, we_s, act_s, bm_s:
                         (we_s[b], 0, imap(b, i, act_s))),
        ],
        out_specs=pl.BlockSpec((TM, H), xmap),
        scratch_shapes=[pltpu.VMEM((TM, H), jnp.bfloat16)],
    )
    return pl.pallas_call(
        _ffn_body,
        grid_spec=grid_spec,
        out_shape=jax.ShapeDtypeStruct((RP, H), jnp.float32),
        compiler_params=pltpu.CompilerParams(
            dimension_semantics=("arbitrary", "arbitrary")),
        interpret=interpret,
    )(we, act, bmap, xs, gate_w, up_w, down_w)


_SCH = 32
_SROWS = R // NW
_SSUB = _SROWS // _SCH


def _scatter_x_body(pos_hbm, x_hbm, xs_hbm, idx_v, buf_a, buf_b, sem_a,
                    sem_b):
    wid = lax.axis_index("s") * NC + lax.axis_index("c")
    pltpu.sync_copy(pos_hbm.at[pl.ds(wid * _SSUB, _SSUB)], idx_v)
    tok0 = (wid % (T // _SROWS)) * _SROWS
    bufs = (buf_a, buf_b)
    sems = (sem_a, sem_b)
    cps = [None, None]
    for c in range(_SSUB):
        if cps[c % 2] is not None:
            cps[c % 2].wait()
        pltpu.sync_copy(x_hbm.at[pl.ds(tok0 + c * _SCH, _SCH)], bufs[c % 2])
        cps[c % 2] = pltpu.async_copy(bufs[c % 2], xs_hbm.at[idx_v.at[c]],
                                      sems[c % 2])
    cps[0].wait()
    cps[1].wait()


def _scatter_x(pos4, x2d):
    kfn = pl.kernel(
        _scatter_x_body,
        out_type=jax.ShapeDtypeStruct((RP, H), jnp.float32),
        mesh=plsc.VectorSubcoreMesh(core_axis_name="c", subcore_axis_name="s"),
        scratch_types=[
            pltpu.VMEM((_SSUB, _SCH), jnp.int32),
            pltpu.VMEM((_SCH, H), jnp.float32),
            pltpu.VMEM((_SCH, H), jnp.float32),
            pltpu.SemaphoreType.DMA,
            pltpu.SemaphoreType.DMA,
        ],
    )
    return kfn(pos4, x2d)


_CCH = 32
_CSUB = T // NW // _CCH


def _combine_body(p0_hbm, p1_hbm, w1_hbm, w2_hbm, y_hbm, out_hbm, idx_a,
                  idx_b, wbuf_a, wbuf_b, buf_a, buf_b, sem):
    wid = lax.axis_index("s") * NC + lax.axis_index("c")
    for s in range(_CSUB):
        row = wid * _CSUB + s
        pltpu.sync_copy(p0_hbm.at[row], idx_a)
        pltpu.sync_copy(p1_hbm.at[row], idx_b)
        pltpu.sync_copy(w1_hbm.at[pl.ds(row * _CCH, _CCH)], wbuf_a)
        pltpu.sync_copy(w2_hbm.at[pl.ds(row * _CCH, _CCH)], wbuf_b)
        pltpu.async_copy(y_hbm.at[idx_a], buf_a, sem).wait()
        pltpu.async_copy(y_hbm.at[idx_b], buf_b, sem).wait()
        for r in range(_CCH):
            wa = wbuf_a[r, :]
            wb = wbuf_b[r, :]

            def inner(jc, _, r=r, wa=wa, wb=wb):
                off = jc * 16
                buf_a[r, pl.ds(off, 16)] = (wa * buf_a[r, pl.ds(off, 16)]
                                            + wb * buf_b[r, pl.ds(off, 16)])
                return 0
            lax.fori_loop(0, H // 16, inner, 0)
        pltpu.sync_copy(buf_a, out_hbm.at[pl.ds(row * _CCH, _CCH)])


def _combine(p0, p1, w1r, w2r, ys):
    kfn = pl.kernel(
        _combine_body,
        out_type=jax.ShapeDtypeStruct((T, H), jnp.float32),
        mesh=plsc.VectorSubcoreMesh(core_axis_name="c", subcore_axis_name="s"),
        scratch_types=[
            pltpu.VMEM((_CCH,), jnp.int32),
            pltpu.VMEM((_CCH,), jnp.int32),
            pltpu.VMEM((_CCH, 16), jnp.float32),
            pltpu.VMEM((_CCH, 16), jnp.float32),
            pltpu.VMEM((_CCH, H), jnp.float32),
            pltpu.VMEM((_CCH, H), jnp.float32),
            pltpu.SemaphoreType.DMA,
        ],
    )
    return kfn(p0, p1, w1r, w2r, ys)


def kernel(x, Wr, gate_w, up_w, down_w):
    Bq, Sq, Hq = x.shape
    x2d = x.reshape(T, H)
    pos, w1r, w2r, we, act, bmap = _dispatch(x2d, Wr)
    pos1 = pos.reshape(R)
    xs = _scatter_x(pos.reshape(NW * _SSUB, _SCH), x2d)
    ys = _ffn(we.reshape(NB), act.reshape(NB), bmap.reshape(NB),
              xs, gate_w, up_w, down_w)
    out = _combine(pos1[:T].reshape(T // _CCH, _CCH),
                   pos1[T:].reshape(T // _CCH, _CCH), w1r, w2r, ys)
    return out.reshape(Bq, Sq, Hq)


# speedup vs baseline: 1.0504x; 1.0504x over previous
import functools

import jax
import jax.numpy as jnp
from jax import lax
from jax.experimental import pallas as pl
from jax.experimental.pallas import tpu as pltpu
from jax.experimental.pallas import tpu_sc as plsc

H = 1024
I = 2816
E = 9
K = 2
T = 2048
R = T * K
TM = 512
TI = 256
NI = I // TI
NB = 16
RP = NB * TM
NC, NS = 2, 16
NW = NC * NS


def _dispatch_body(x_ref, wr_ref, pos_ref, w1_ref, w2_ref, we_ref, act_ref,
                   bmap_ref):
    xr = x_ref[...]
    wr = wr_ref[...]
    logits = lax.dot_general(xr, wr, (((1,), (1,)), ((), ())),
                             preferred_element_type=jnp.float32)
    iota_e = lax.broadcasted_iota(jnp.int32, (T, E), 1)
    m1 = jnp.max(logits, axis=1, keepdims=True)
    a1 = jnp.min(jnp.where(logits == m1, iota_e, E), axis=1, keepdims=True)
    neg = jnp.where(iota_e == a1, -jnp.inf, logits)
    m2 = jnp.max(neg, axis=1, keepdims=True)
    a2 = jnp.min(jnp.where(neg == m2, iota_e, E), axis=1, keepdims=True)
    tt = jnp.exp(m2 - m1)
    w1 = 1.0 / (1.0 + tt)
    w2 = 1.0 - w1
    ev = jnp.concatenate([a1, a2], axis=0)
    oh = (ev == lax.broadcasted_iota(jnp.int32, (R, E), 1)).astype(jnp.float32)
    C = 512
    ci = lax.broadcasted_iota(jnp.int32, (C, C), 0)
    cj = lax.broadcasted_iota(jnp.int32, (C, C), 1)
    lmat = (ci > cj).astype(jnp.float32)
    off = jnp.zeros((1, E), jnp.float32)
    ranks = []
    for c in range(R // C):
        ohc = lax.slice(oh, (c * C, 0), ((c + 1) * C, E))
        loc = lax.dot_general(lmat, ohc, (((1,), (0,)), ((), ())),
                              preferred_element_type=jnp.float32)
        ranks.append(jnp.sum(ohc * (loc + off), axis=1, keepdims=True))
        off = off + lax.slice(loc + ohc, (C - 1, 0), (C, E))
    rank = jnp.concatenate(ranks, axis=0)
    counts = off
    nb = jnp.floor((counts + (TM - 1)) / TM)
    ei = lax.broadcasted_iota(jnp.int32, (E, E), 0)
    ej = lax.broadcasted_iota(jnp.int32, (E, E), 1)
    tmat = (ei < ej).astype(jnp.float32)
    esum = lax.dot_general(nb, tmat, (((1,), (0,)), ((), ())),
                           preferred_element_type=jnp.float32)
    start = esum * TM
    posf = jnp.sum(oh * start, axis=1, keepdims=True) + rank
    pos_ref[...] = posf.astype(jnp.int32)
    w1_ref[...] = jnp.broadcast_to(w1, (T, 16))
    w2_ref[...] = jnp.broadcast_to(w2, (T, 16))
    tot = lax.slice(esum + nb, (0, E - 1), (1, E))
    bio = lax.broadcasted_iota(jnp.int32, (NB, 1), 0).astype(jnp.float32)
    act = (bio < tot).astype(jnp.int32)
    bcl = jnp.minimum(bio, tot - 1.0)
    cmp = (esum <= bcl).astype(jnp.float32)
    be = jnp.sum(cmp, axis=1, keepdims=True) - 1.0
    we_ref[...] = be.astype(jnp.int32)
    act_ref[...] = act
    bmap_ref[...] = bcl.astype(jnp.int32)


def _dispatch(x2d, wr, interpret=False):
    outs = (
        jax.ShapeDtypeStruct((R, 1), jnp.int32),
        jax.ShapeDtypeStruct((T, 16), jnp.float32),
        jax.ShapeDtypeStruct((T, 16), jnp.float32),
        jax.ShapeDtypeStruct((NB, 1), jnp.int32),
        jax.ShapeDtypeStruct((NB, 1), jnp.int32),
        jax.ShapeDtypeStruct((NB, 1), jnp.int32),
    )
    return pl.pallas_call(_dispatch_body, out_shape=outs,
                          interpret=interpret)(x2d, wr)


def _ffn_body(we_s, act_s, bmap_s, x_blk, g_blk, u_blk, d_blk, y_blk):
    b = pl.program_id(0)
    i = pl.program_id(1)

    @pl.when(act_s[b] == 1)
    def _():
        x = x_blk[...]
        g = lax.dot_general(x, g_blk[0], (((1,), (1,)), ((), ())),
                            preferred_element_type=jnp.float32)
        u = lax.dot_general(x, u_blk[0], (((1,), (1,)), ((), ())),
                            preferred_element_type=jnp.float32)
        h = g * (1.0 / (1.0 + jnp.exp(-g))) * u
        contrib = lax.dot_general(h, d_blk[0], (((1,), (1,)), ((), ())),
                                  preferred_element_type=jnp.float32)

        @pl.when(i == 0)
        def _():
            y_blk[...] = contrib

        @pl.when(i != 0)
        def _():
            y_blk[...] = y_blk[...] + contrib


def _ffn(we, act, bmap, xs, gate_w, up_w, down_w, interpret=False):
    def xmap(b, i, we_s, act_s, bm_s):
        return (bm_s[b], 0)

    def imap(b, i, act_s):
        return jnp.where(act_s[b] == 1, i, NI - 1)

    grid_spec = pltpu.PrefetchScalarGridSpec(
        num_scalar_prefetch=3,
        grid=(NB, NI),
        in_specs=[
            pl.BlockSpec((TM, H), xmap),
            pl.BlockSpec((1, TI, H),
                         lambda b, i, we_s, act_s, bm_s:
                         (we_s[b], imap(b, i, act_s), 0)),
            pl.BlockSpec((1, TI, H),
                         lambda b, i, we_s, act_s, bm_s:
                         (we_s[b], imap(b, i, act_s), 0)),
            pl.BlockSpec((1, H, TI),
                         lambda b, i, we_s, act_s, bm_s:
                         (we_s[b], 0, imap(b, i, act_s))),
        ],
        out_specs=pl.BlockSpec((TM, H), xmap),
    )
    return pl.pallas_call(
        _ffn_body,
        grid_spec=grid_spec,
        out_shape=jax.ShapeDtypeStruct((RP, H), jnp.float32),
        compiler_params=pltpu.CompilerParams(
            dimension_semantics=("arbitrary", "arbitrary")),
        interpret=interpret,
    )(we, act, bmap, xs, gate_w, up_w, down_w)


_SCH = 32
_SROWS = R // NW
_SSUB = _SROWS // _SCH


def _scatter_x_body(pos_hbm, x_hbm, xs_hbm, idx_v, buf_a, buf_b, sem_a,
                    sem_b):
    wid = lax.axis_index("s") * NC + lax.axis_index("c")
    pltpu.sync_copy(pos_hbm.at[pl.ds(wid * _SSUB, _SSUB)], idx_v)
    tok0 = (wid % (T // _SROWS)) * _SROWS
    bufs = (buf_a, buf_b)
    sems = (sem_a, sem_b)
    cps = [None, None]
    for c in range(_SSUB):
        if cps[c % 2] is not None:
            cps[c % 2].wait()
        pltpu.sync_copy(x_hbm.at[pl.ds(tok0 + c * _SCH, _SCH)], bufs[c % 2])
        cps[c % 2] = pltpu.async_copy(bufs[c % 2], xs_hbm.at[idx_v.at[c]],
                                      sems[c % 2])
    cps[0].wait()
    cps[1].wait()


def _scatter_x(pos4, x2d):
    kfn = pl.kernel(
        _scatter_x_body,
        out_type=jax.ShapeDtypeStruct((RP, H), jnp.float32),
        mesh=plsc.VectorSubcoreMesh(core_axis_name="c", subcore_axis_name="s"),
        scratch_types=[
            pltpu.VMEM((_SSUB, _SCH), jnp.int32),
            pltpu.VMEM((_SCH, H), jnp.float32),
            pltpu.VMEM((_SCH, H), jnp.float32),
            pltpu.SemaphoreType.DMA,
            pltpu.SemaphoreType.DMA,
        ],
    )
    return kfn(pos4, x2d)


_CCH = 16
_CSUB = T // NW // _CCH
_CW = T // NW


def _combine_body(p0_hbm, p1_hbm, w1_hbm, w2_hbm, y_hbm, out_hbm, idx_a,
                  idx_b, wbuf_a, wbuf_b, buf_a, buf_b, sem_a, sem_b, sem_o):
    wid = lax.axis_index("s") * NC + lax.axis_index("c")
    pltpu.sync_copy(p0_hbm.at[pl.ds(wid * _CSUB, _CSUB)], idx_a)
    pltpu.sync_copy(p1_hbm.at[pl.ds(wid * _CSUB, _CSUB)], idx_b)
    pltpu.sync_copy(w1_hbm.at[pl.ds(wid * _CW, _CW)], wbuf_a)
    pltpu.sync_copy(w2_hbm.at[pl.ds(wid * _CW, _CW)], wbuf_b)
    cpa = [None, None]
    cpb = [None, None]
    cpo = [None, None]

    def issue(s, p):
        cpa[p] = pltpu.async_copy(y_hbm.at[idx_a.at[s]], buf_a.at[p], sem_a[p])
        cpb[p] = pltpu.async_copy(y_hbm.at[idx_b.at[s]], buf_b.at[p], sem_b[p])

    issue(0, 0)
    for s in range(_CSUB):
        p = s % 2
        if s + 1 < _CSUB:
            if cpo[1 - p] is not None:
                cpo[1 - p].wait()
            issue(s + 1, 1 - p)
        cpa[p].wait()
        cpb[p].wait()
        for r in range(_CCH):
            wa = wbuf_a[s * _CCH + r, :]
            wb = wbuf_b[s * _CCH + r, :]

            def inner(jc, _, r=r, wa=wa, wb=wb, p=p):
                off = jc * 64
                for q in range(4):
                    o = off + q * 16
                    buf_a[p, r, pl.ds(o, 16)] = (
                        wa * buf_a[p, r, pl.ds(o, 16)]
                        + wb * buf_b[p, r, pl.ds(o, 16)])
                return 0
            lax.fori_loop(0, H // 64, inner, 0)
        cpo[p] = pltpu.async_copy(
            buf_a.at[p], out_hbm.at[pl.ds(wid * _CW + s * _CCH, _CCH)],
            sem_o[p])
    cpo[0].wait()
    cpo[1].wait()


def _combine(p0, p1, w1r, w2r, ys):
    kfn = pl.kernel(
        _combine_body,
        out_type=jax.ShapeDtypeStruct((T, H), jnp.float32),
        mesh=plsc.VectorSubcoreMesh(core_axis_name="c", subcore_axis_name="s"),
        scratch_types=[
            pltpu.VMEM((_CSUB, _CCH), jnp.int32),
            pltpu.VMEM((_CSUB, _CCH), jnp.int32),
            pltpu.VMEM((_CW, 16), jnp.float32),
            pltpu.VMEM((_CW, 16), jnp.float32),
            pltpu.VMEM((2, _CCH, H), jnp.float32),
            pltpu.VMEM((2, _CCH, H), jnp.float32),
            [pltpu.SemaphoreType.DMA, pltpu.SemaphoreType.DMA],
            [pltpu.SemaphoreType.DMA, pltpu.SemaphoreType.DMA],
            [pltpu.SemaphoreType.DMA, pltpu.SemaphoreType.DMA],
        ],
    )
    return kfn(p0, p1, w1r, w2r, ys)


def kernel(x, Wr, gate_w, up_w, down_w):
    Bq, Sq, Hq = x.shape
    x2d = x.reshape(T, H)
    pos, w1r, w2r, we, act, bmap = _dispatch(x2d, Wr)
    pos1 = pos.reshape(R)
    xs = _scatter_x(pos.reshape(NW * _SSUB, _SCH), x2d)
    ys = _ffn(we.reshape(NB), act.reshape(NB), bmap.reshape(NB),
              xs, gate_w, up_w, down_w)
    out = _combine(pos1[:T].reshape(T // _CCH, _CCH),
                   pos1[T:].reshape(T // _CCH, _CCH), w1r, w2r, ys)
    return out.reshape(Bq, Sq, Hq)


# speedup vs baseline: 1.0871x; 1.0350x over previous
import functools

import jax
import jax.numpy as jnp
from jax import lax
from jax.experimental import pallas as pl
from jax.experimental.pallas import tpu as pltpu
from jax.experimental.pallas import tpu_sc as plsc

H = 1024
I = 2816
E = 9
K = 2
T = 2048
R = T * K
TM = 512
TI = 256
NI = I // TI
NB = 16
RP = NB * TM
NC, NS = 2, 16
NW = NC * NS


def _dispatch_body(x_ref, wr_ref, pos_ref, w1_ref, w2_ref, we_ref, act_ref,
                   bmap_ref):
    xr = x_ref[...]
    wr = wr_ref[...]
    logits = lax.dot_general(xr, wr, (((1,), (1,)), ((), ())),
                             preferred_element_type=jnp.float32)
    iota_e = lax.broadcasted_iota(jnp.int32, (T, E), 1)
    m1 = jnp.max(logits, axis=1, keepdims=True)
    a1 = jnp.min(jnp.where(logits == m1, iota_e, E), axis=1, keepdims=True)
    neg = jnp.where(iota_e == a1, -jnp.inf, logits)
    m2 = jnp.max(neg, axis=1, keepdims=True)
    a2 = jnp.min(jnp.where(neg == m2, iota_e, E), axis=1, keepdims=True)
    tt = jnp.exp(m2 - m1)
    w1 = 1.0 / (1.0 + tt)
    w2 = 1.0 - w1
    ev = jnp.concatenate([a1, a2], axis=0)
    oh = (ev == lax.broadcasted_iota(jnp.int32, (R, E), 1)).astype(jnp.float32)
    C = 512
    ci = lax.broadcasted_iota(jnp.int32, (C, C), 0)
    cj = lax.broadcasted_iota(jnp.int32, (C, C), 1)
    lmat = (ci > cj).astype(jnp.float32)
    off = jnp.zeros((1, E), jnp.float32)
    ranks = []
    for c in range(R // C):
        ohc = lax.slice(oh, (c * C, 0), ((c + 1) * C, E))
        loc = lax.dot_general(lmat, ohc, (((1,), (0,)), ((), ())),
                              preferred_element_type=jnp.float32)
        ranks.append(jnp.sum(ohc * (loc + off), axis=1, keepdims=True))
        off = off + lax.slice(loc + ohc, (C - 1, 0), (C, E))
    rank = jnp.concatenate(ranks, axis=0)
    counts = off
    nb = jnp.floor((counts + (TM - 1)) / TM)
    ei = lax.broadcasted_iota(jnp.int32, (E, E), 0)
    ej = lax.broadcasted_iota(jnp.int32, (E, E), 1)
    tmat = (ei < ej).astype(jnp.float32)
    esum = lax.dot_general(nb, tmat, (((1,), (0,)), ((), ())),
                           preferred_element_type=jnp.float32)
    start = esum * TM
    posf = jnp.sum(oh * start, axis=1, keepdims=True) + rank
    pos_ref[...] = posf.astype(jnp.int32)
    w1_ref[...] = jnp.broadcast_to(w1, (T, 16))
    w2_ref[...] = jnp.broadcast_to(w2, (T, 16))
    tot = lax.slice(esum + nb, (0, E - 1), (1, E))
    bio = lax.broadcasted_iota(jnp.int32, (NB, 1), 0).astype(jnp.float32)
    act = (bio < tot).astype(jnp.int32)
    bcl = jnp.minimum(bio, tot - 1.0)
    cmp = (esum <= bcl).astype(jnp.float32)
    be = jnp.sum(cmp, axis=1, keepdims=True) - 1.0
    we_ref[...] = be.astype(jnp.int32)
    act_ref[...] = act
    bmap_ref[...] = bcl.astype(jnp.int32)


def _dispatch(x2d, wr, interpret=False):
    outs = (
        jax.ShapeDtypeStruct((R, 1), jnp.int32),
        jax.ShapeDtypeStruct((T, 16), jnp.float32),
        jax.ShapeDtypeStruct((T, 16), jnp.float32),
        jax.ShapeDtypeStruct((NB, 1), jnp.int32),
        jax.ShapeDtypeStruct((NB, 1), jnp.int32),
        jax.ShapeDtypeStruct((NB, 1), jnp.int32),
    )
    return pl.pallas_call(_dispatch_body, out_shape=outs,
                          interpret=interpret)(x2d, wr)


def _ffn_body(we_s, act_s, bmap_s, x_blk, g_blk, u_blk, d_blk, y_blk):
    b = pl.program_id(0)
    i = pl.program_id(1)

    @pl.when(act_s[b] == 1)
    def _():
        x = x_blk[...]
        g = lax.dot_general(x, g_blk[0], (((1,), (1,)), ((), ())),
                            preferred_element_type=jnp.float32)
        u = lax.dot_general(x, u_blk[0], (((1,), (1,)), ((), ())),
                            preferred_element_type=jnp.float32)
        h = g * (1.0 / (1.0 + jnp.exp(-g))) * u
        contrib = lax.dot_general(h, d_blk[0], (((1,), (1,)), ((), ())),
                                  preferred_element_type=jnp.float32)

        @pl.when(i == 0)
        def _():
            y_blk[...] = contrib

        @pl.when(i != 0)
        def _():
            y_blk[...] = y_blk[...] + contrib


def _ffn(we, act, bmap, xs, gate_w, up_w, down_w, interpret=False):
    def xmap(b, i, we_s, act_s, bm_s):
        return (bm_s[b], 0)

    def imap(b, i, act_s):
        return jnp.where(act_s[b] == 1, i, NI - 1)

    grid_spec = pltpu.PrefetchScalarGridSpec(
        num_scalar_prefetch=3,
        grid=(NB, NI),
        in_specs=[
            pl.BlockSpec((TM, H), xmap),
            pl.BlockSpec((1, TI, H),
                         lambda b, i, we_s, act_s, bm_s:
                         (we_s[b], imap(b, i, act_s), 0)),
            pl.BlockSpec((1, TI, H),
                         lambda b, i, we_s, act_s, bm_s:
                         (we_s[b], imap(b, i, act_s), 0)),
            pl.BlockSpec((1, H, TI),
                         lambda b, i, we_s, act_s, bm_s:
                         (we_s[b], 0, imap(b, i, act_s))),
        ],
        out_specs=pl.BlockSpec((TM, H), xmap),
    )
    return pl.pallas_call(
        _ffn_body,
        grid_spec=grid_spec,
        out_shape=jax.ShapeDtypeStruct((RP, H), jnp.float32),
        compiler_params=pltpu.CompilerParams(
            dimension_semantics=("arbitrary", "arbitrary")),
        interpret=interpret,
    )(we, act, bmap, xs, gate_w, up_w, down_w)


_SCH = 32
_SROWS = R // NW
_SSUB = _SROWS // _SCH


def _scatter_x_body(pos_hbm, x_hbm, xs_hbm, idx_v, buf, sem_l, sem_s):
    wid = lax.axis_index("s") * NC + lax.axis_index("c")
    pltpu.sync_copy(pos_hbm.at[pl.ds(wid * _SSUB, _SSUB)], idx_v)
    tok0 = (wid % (T // _SROWS)) * _SROWS
    lds = [None, None]
    sts = [None, None]

    def load(c, p):
        lds[p] = pltpu.async_copy(x_hbm.at[pl.ds(tok0 + c * _SCH, _SCH)],
                                  buf.at[p], sem_l[p])

    load(0, 0)
    for c in range(_SSUB):
        p = c % 2
        if c + 1 < _SSUB:
            if sts[1 - p] is not None:
                sts[1 - p].wait()
            load(c + 1, 1 - p)
        lds[p].wait()
        sts[p] = pltpu.async_copy(buf.at[p], xs_hbm.at[idx_v.at[c]], sem_s[p])
    sts[0].wait()
    sts[1].wait()


def _scatter_x(pos4, x2d):
    kfn = pl.kernel(
        _scatter_x_body,
        out_type=jax.ShapeDtypeStruct((RP, H), jnp.float32),
        mesh=plsc.VectorSubcoreMesh(core_axis_name="c", subcore_axis_name="s"),
        scratch_types=[
            pltpu.VMEM((_SSUB, _SCH), jnp.int32),
            pltpu.VMEM((2, _SCH, H), jnp.float32),
            [pltpu.SemaphoreType.DMA, pltpu.SemaphoreType.DMA],
            [pltpu.SemaphoreType.DMA, pltpu.SemaphoreType.DMA],
        ],
    )
    return kfn(pos4, x2d)


_CCH = 16
_CSUB = T // NW // _CCH
_CW = T // NW


def _combine_body(p0_hbm, p1_hbm, w1_hbm, w2_hbm, y_hbm, out_hbm, idx_a,
                  idx_b, wbuf_a, wbuf_b, buf_a, buf_b, sem_a, sem_b, sem_o):
    wid = lax.axis_index("s") * NC + lax.axis_index("c")
    pltpu.sync_copy(p0_hbm.at[pl.ds(wid * _CSUB, _CSUB)], idx_a)
    pltpu.sync_copy(p1_hbm.at[pl.ds(wid * _CSUB, _CSUB)], idx_b)
    pltpu.sync_copy(w1_hbm.at[pl.ds(wid * _CW, _CW)], wbuf_a)
    pltpu.sync_copy(w2_hbm.at[pl.ds(wid * _CW, _CW)], wbuf_b)
    cpa = [None, None]
    cpb = [None, None]
    cpo = [None, None]

    def issue(s, p):
        cpa[p] = pltpu.async_copy(y_hbm.at[idx_a.at[s]], buf_a.at[p], sem_a[p])
        cpb[p] = pltpu.async_copy(y_hbm.at[idx_b.at[s]], buf_b.at[p], sem_b[p])

    issue(0, 0)
    for s in range(_CSUB):
        p = s % 2
        if s + 1 < _CSUB:
            if cpo[1 - p] is not None:
                cpo[1 - p].wait()
            issue(s + 1, 1 - p)
        cpa[p].wait()
        cpb[p].wait()
        for r in range(_CCH):
            wa = wbuf_a[s * _CCH + r, :]
            wb = wbuf_b[s * _CCH + r, :]

            def inner(jc, _, r=r, wa=wa, wb=wb, p=p):
                off = jc * 256
                for q in range(16):
                    o = off + q * 16
                    buf_a[p, r, pl.ds(o, 16)] = (
                        wa * buf_a[p, r, pl.ds(o, 16)]
                        + wb * buf_b[p, r, pl.ds(o, 16)])
                return 0
            lax.fori_loop(0, H // 256, inner, 0)
        cpo[p] = pltpu.async_copy(
            buf_a.at[p], out_hbm.at[pl.ds(wid * _CW + s * _CCH, _CCH)],
            sem_o[p])
    cpo[0].wait()
    cpo[1].wait()


def _combine(p0, p1, w1r, w2r, ys):
    kfn = pl.kernel(
        _combine_body,
        out_type=jax.ShapeDtypeStruct((T, H), jnp.float32),
        mesh=plsc.VectorSubcoreMesh(core_axis_name="c", subcore_axis_name="s"),
        scratch_types=[
            pltpu.VMEM((_CSUB, _CCH), jnp.int32),
            pltpu.VMEM((_CSUB, _CCH), jnp.int32),
            pltpu.VMEM((_CW, 16), jnp.float32),
            pltpu.VMEM((_CW, 16), jnp.float32),
            pltpu.VMEM((2, _CCH, H), jnp.float32),
            pltpu.VMEM((2, _CCH, H), jnp.float32),
            [pltpu.SemaphoreType.DMA, pltpu.SemaphoreType.DMA],
            [pltpu.SemaphoreType.DMA, pltpu.SemaphoreType.DMA],
            [pltpu.SemaphoreType.DMA, pltpu.SemaphoreType.DMA],
        ],
    )
    return kfn(p0, p1, w1r, w2r, ys)


def kernel(x, Wr, gate_w, up_w, down_w):
    Bq, Sq, Hq = x.shape
    x2d = x.reshape(T, H)
    pos, w1r, w2r, we, act, bmap = _dispatch(x2d, Wr)
    pos1 = pos.reshape(R)
    xs = _scatter_x(pos.reshape(NW * _SSUB, _SCH), x2d)
    ys = _ffn(we.reshape(NB), act.reshape(NB), bmap.reshape(NB),
              xs, gate_w, up_w, down_w)
    out = _combine(pos1[:T].reshape(T // _CCH, _CCH),
                   pos1[T:].reshape(T // _CCH, _CCH), w1r, w2r, ys)
    return out.reshape(Bq, Sq, Hq)


# speedup vs baseline: 1.1123x; 1.0232x over previous
import functools

import jax
import jax.numpy as jnp
from jax import lax
from jax.experimental import pallas as pl
from jax.experimental.pallas import tpu as pltpu
from jax.experimental.pallas import tpu_sc as plsc

H = 1024
I = 2816
E = 9
K = 2
T = 2048
R = T * K
TM = 512
TI = 256
NI = I // TI
NB = 16
RP = NB * TM
NC, NS = 2, 16
NW = NC * NS


_DCH = 512


def _dispatch_body(x_ref, wr_ref, pos_ref, w1_ref, w2_ref, we_ref, act_ref,
                   bmap_ref, a1_s, a2_s):
    k = pl.program_id(0)
    xr = x_ref[...]
    wr = wr_ref[...]
    logits = lax.dot_general(xr, wr, (((1,), (1,)), ((), ())),
                             preferred_element_type=jnp.float32)
    iota_e = lax.broadcasted_iota(jnp.int32, (_DCH, E), 1)
    m1 = jnp.max(logits, axis=1, keepdims=True)
    a1 = jnp.min(jnp.where(logits == m1, iota_e, E), axis=1, keepdims=True)
    neg = jnp.where(iota_e == a1, -jnp.inf, logits)
    m2 = jnp.max(neg, axis=1, keepdims=True)
    a2 = jnp.min(jnp.where(neg == m2, iota_e, E), axis=1, keepdims=True)
    tt = jnp.exp(m2 - m1)
    w1 = 1.0 / (1.0 + tt)
    w2 = 1.0 - w1
    a1_s[pl.ds(k * _DCH, _DCH), :] = a1
    a2_s[pl.ds(k * _DCH, _DCH), :] = a2
    w1_ref[...] = jnp.broadcast_to(w1, (_DCH, 16))
    w2_ref[...] = jnp.broadcast_to(w2, (_DCH, 16))

    @pl.when(k == T // _DCH - 1)
    def _():
        _dispatch_tail(pos_ref, we_ref, act_ref, bmap_ref, a1_s, a2_s)


def _dispatch_tail(pos_ref, we_ref, act_ref, bmap_ref, a1_s, a2_s):
    ev = jnp.concatenate([a1_s[...], a2_s[...]], axis=0)
    oh = (ev == lax.broadcasted_iota(jnp.int32, (R, E), 1)).astype(jnp.float32)
    C = 512
    ci = lax.broadcasted_iota(jnp.int32, (C, C), 0)
    cj = lax.broadcasted_iota(jnp.int32, (C, C), 1)
    lmat = (ci > cj).astype(jnp.float32)
    off = jnp.zeros((1, E), jnp.float32)
    ranks = []
    for c in range(R // C):
        ohc = lax.slice(oh, (c * C, 0), ((c + 1) * C, E))
        loc = lax.dot_general(lmat, ohc, (((1,), (0,)), ((), ())),
                              preferred_element_type=jnp.float32)
        ranks.append(jnp.sum(ohc * (loc + off), axis=1, keepdims=True))
        off = off + lax.slice(loc + ohc, (C - 1, 0), (C, E))
    rank = jnp.concatenate(ranks, axis=0)
    counts = off
    nb = jnp.floor((counts + (TM - 1)) / TM)
    ei = lax.broadcasted_iota(jnp.int32, (E, E), 0)
    ej = lax.broadcasted_iota(jnp.int32, (E, E), 1)
    tmat = (ei < ej).astype(jnp.float32)
    esum = lax.dot_general(nb, tmat, (((1,), (0,)), ((), ())),
                           preferred_element_type=jnp.float32)
    start = esum * TM
    posf = jnp.sum(oh * start, axis=1, keepdims=True) + rank
    pos_ref[...] = posf.astype(jnp.int32)
    tot = lax.slice(esum + nb, (0, E - 1), (1, E))
    bio = lax.broadcasted_iota(jnp.int32, (NB, 1), 0).astype(jnp.float32)
    act = (bio < tot).astype(jnp.int32)
    bcl = jnp.minimum(bio, tot - 1.0)
    cmp = (esum <= bcl).astype(jnp.float32)
    be = jnp.sum(cmp, axis=1, keepdims=True) - 1.0
    we_ref[...] = be.astype(jnp.int32)
    act_ref[...] = act
    bmap_ref[...] = bcl.astype(jnp.int32)


def _dispatch(x2d, wr, interpret=False):
    outs = (
        jax.ShapeDtypeStruct((R, 1), jnp.int32),
        jax.ShapeDtypeStruct((T, 16), jnp.float32),
        jax.ShapeDtypeStruct((T, 16), jnp.float32),
        jax.ShapeDtypeStruct((NB, 1), jnp.int32),
        jax.ShapeDtypeStruct((NB, 1), jnp.int32),
        jax.ShapeDtypeStruct((NB, 1), jnp.int32),
    )
    nk = T // _DCH
    return pl.pallas_call(
        _dispatch_body,
        grid=(nk,),
        in_specs=[
            pl.BlockSpec((_DCH, H), lambda k: (k, 0)),
            pl.BlockSpec((E, H), lambda k: (0, 0)),
        ],
        out_specs=(
            pl.BlockSpec((R, 1), lambda k: (0, 0)),
            pl.BlockSpec((_DCH, 16), lambda k: (k, 0)),
            pl.BlockSpec((_DCH, 16), lambda k: (k, 0)),
            pl.BlockSpec((NB, 1), lambda k: (0, 0)),
            pl.BlockSpec((NB, 1), lambda k: (0, 0)),
            pl.BlockSpec((NB, 1), lambda k: (0, 0)),
        ),
        scratch_shapes=[pltpu.VMEM((T, 1), jnp.int32),
                        pltpu.VMEM((T, 1), jnp.int32)],
        out_shape=outs,
        compiler_params=pltpu.CompilerParams(
            dimension_semantics=("arbitrary",)),
        interpret=interpret)(x2d, wr)


def _ffn_body(we_s, act_s, bmap_s, x_blk, g_blk, u_blk, d_blk, y_blk):
    b = pl.program_id(0)
    i = pl.program_id(1)

    @pl.when(act_s[b] == 1)
    def _():
        x = x_blk[...]
        g = lax.dot_general(x, g_blk[0], (((1,), (1,)), ((), ())),
                            preferred_element_type=jnp.float32)
        u = lax.dot_general(x, u_blk[0], (((1,), (1,)), ((), ())),
                            preferred_element_type=jnp.float32)
        h = g * (1.0 / (1.0 + jnp.exp(-g))) * u
        contrib = lax.dot_general(h, d_blk[0], (((1,), (1,)), ((), ())),
                                  preferred_element_type=jnp.float32)

        @pl.when(i == 0)
        def _():
            y_blk[...] = contrib

        @pl.when(i != 0)
        def _():
            y_blk[...] = y_blk[...] + contrib


def _ffn(we, act, bmap, xs, gate_w, up_w, down_w, interpret=False):
    def xmap(b, i, we_s, act_s, bm_s):
        return (bm_s[b], 0)

    def imap(b, i, act_s):
        return jnp.where(act_s[b] == 1, i, NI - 1)

    grid_spec = pltpu.PrefetchScalarGridSpec(
        num_scalar_prefetch=3,
        grid=(NB, NI),
        in_specs=[
            pl.BlockSpec((TM, H), xmap),
            pl.BlockSpec((1, TI, H),
                         lambda b, i, we_s, act_s, bm_s:
                         (we_s[b], imap(b, i, act_s), 0)),
            pl.BlockSpec((1, TI, H),
                         lambda b, i, we_s, act_s, bm_s:
                         (we_s[b], imap(b, i, act_s), 0)),
            pl.BlockSpec((1, H, TI),
                         lambda b, i, we_s, act_s, bm_s:
                         (we_s[b], 0, imap(b, i, act_s))),
        ],
        out_specs=pl.BlockSpec((TM, H), xmap),
    )
    return pl.pallas_call(
        _ffn_body,
        grid_spec=grid_spec,
        out_shape=jax.ShapeDtypeStruct((RP, H), jnp.float32),
        compiler_params=pltpu.CompilerParams(
            dimension_semantics=("arbitrary", "arbitrary")),
        interpret=interpret,
    )(we, act, bmap, xs, gate_w, up_w, down_w)


_SCH = 32
_SROWS = R // NW
_SSUB = _SROWS // _SCH


def _scatter_x_body(pos_hbm, x_hbm, xs_hbm, idx_v, buf, sem_l, sem_s):
    wid = lax.axis_index("s") * NC + lax.axis_index("c")
    pltpu.sync_copy(pos_hbm.at[pl.ds(wid * _SSUB, _SSUB)], idx_v)
    tok0 = (wid % (T // _SROWS)) * _SROWS
    lds = [None, None]
    sts = [None, None]

    def load(c, p):
        lds[p] = pltpu.async_copy(x_hbm.at[pl.ds(tok0 + c * _SCH, _SCH)],
                                  buf.at[p], sem_l[p])

    load(0, 0)
    for c in range(_SSUB):
        p = c % 2
        if c + 1 < _SSUB:
            if sts[1 - p] is not None:
                sts[1 - p].wait()
            load(c + 1, 1 - p)
        lds[p].wait()
        sts[p] = pltpu.async_copy(buf.at[p], xs_hbm.at[idx_v.at[c]], sem_s[p])
    sts[0].wait()
    sts[1].wait()


def _scatter_x(pos4, x2d):
    kfn = pl.kernel(
        _scatter_x_body,
        out_type=jax.ShapeDtypeStruct((RP, H), jnp.float32),
        mesh=plsc.VectorSubcoreMesh(core_axis_name="c", subcore_axis_name="s"),
        scratch_types=[
            pltpu.VMEM((_SSUB, _SCH), jnp.int32),
            pltpu.VMEM((2, _SCH, H), jnp.float32),
            [pltpu.SemaphoreType.DMA, pltpu.SemaphoreType.DMA],
            [pltpu.SemaphoreType.DMA, pltpu.SemaphoreType.DMA],
        ],
    )
    return kfn(pos4, x2d)


_CCH = 16
_CSUB = T // NW // _CCH
_CW = T // NW


def _combine_body(p0_hbm, p1_hbm, w1_hbm, w2_hbm, y_hbm, out_hbm, idx_a,
                  idx_b, wbuf_a, wbuf_b, buf_a, buf_b, sem_a, sem_b, sem_o):
    wid = lax.axis_index("s") * NC + lax.axis_index("c")
    pltpu.sync_copy(p0_hbm.at[pl.ds(wid * _CSUB, _CSUB)], idx_a)
    pltpu.sync_copy(p1_hbm.at[pl.ds(wid * _CSUB, _CSUB)], idx_b)
    pltpu.sync_copy(w1_hbm.at[pl.ds(wid * _CW, _CW)], wbuf_a)
    pltpu.sync_copy(w2_hbm.at[pl.ds(wid * _CW, _CW)], wbuf_b)
    cpa = [None, None]
    cpb = [None, None]
    cpo = [None, None]

    def issue(s, p):
        cpa[p] = pltpu.async_copy(y_hbm.at[idx_a.at[s]], buf_a.at[p], sem_a[p])
        cpb[p] = pltpu.async_copy(y_hbm.at[idx_b.at[s]], buf_b.at[p], sem_b[p])

    issue(0, 0)
    for s in range(_CSUB):
        p = s % 2
        if s + 1 < _CSUB:
            if cpo[1 - p] is not None:
                cpo[1 - p].wait()
            issue(s + 1, 1 - p)
        cpa[p].wait()
        cpb[p].wait()
        for r in range(_CCH):
            wa = wbuf_a[s * _CCH + r, :]
            wb = wbuf_b[s * _CCH + r, :]

            def inner(jc, _, r=r, wa=wa, wb=wb, p=p):
                off = jc * 256
                for q in range(16):
                    o = off + q * 16
                    buf_a[p, r, pl.ds(o, 16)] = (
                        wa * buf_a[p, r, pl.ds(o, 16)]
                        + wb * buf_b[p, r, pl.ds(o, 16)])
                return 0
            lax.fori_loop(0, H // 256, inner, 0)
        cpo[p] = pltpu.async_copy(
            buf_a.at[p], out_hbm.at[pl.ds(wid * _CW + s * _CCH, _CCH)],
            sem_o[p])
    cpo[0].wait()
    cpo[1].wait()


def _combine(p0, p1, w1r, w2r, ys):
    kfn = pl.kernel(
        _combine_body,
        out_type=jax.ShapeDtypeStruct((T, H), jnp.float32),
        mesh=plsc.VectorSubcoreMesh(core_axis_name="c", subcore_axis_name="s"),
        scratch_types=[
            pltpu.VMEM((_CSUB, _CCH), jnp.int32),
            pltpu.VMEM((_CSUB, _CCH), jnp.int32),
            pltpu.VMEM((_CW, 16), jnp.float32),
            pltpu.VMEM((_CW, 16), jnp.float32),
            pltpu.VMEM((2, _CCH, H), jnp.float32),
            pltpu.VMEM((2, _CCH, H), jnp.float32),
            [pltpu.SemaphoreType.DMA, pltpu.SemaphoreType.DMA],
            [pltpu.SemaphoreType.DMA, pltpu.SemaphoreType.DMA],
            [pltpu.SemaphoreType.DMA, pltpu.SemaphoreType.DMA],
        ],
    )
    return kfn(p0, p1, w1r, w2r, ys)


def kernel(x, Wr, gate_w, up_w, down_w):
    Bq, Sq, Hq = x.shape
    x2d = x.reshape(T, H)
    pos, w1r, w2r, we, act, bmap = _dispatch(x2d, Wr)
    pos1 = pos.reshape(R)
    xs = _scatter_x(pos.reshape(NW * _SSUB, _SCH), x2d)
    ys = _ffn(we.reshape(NB), act.reshape(NB), bmap.reshape(NB),
              xs, gate_w, up_w, down_w)
    out = _combine(pos1[:T].reshape(T // _CCH, _CCH),
                   pos1[T:].reshape(T // _CCH, _CCH), w1r, w2r, ys)
    return out.reshape(Bq, Sq, Hq)
